# baseline (speedup 1.0000x reference)
import functools

import jax
import jax.numpy as jnp
from jax import lax
from jax.experimental import pallas as pl
from jax.experimental.pallas import tpu as pltpu
from jax.experimental.pallas import tpu_sc as plsc

_N = 10000
_E = 320000
_D = 128
_H = 64
_H2 = 32
_WA = 128
_NC = 2
_NS = 16
_CH = 128
_RCH = _E // _CH
_CPW = 80
_G = 24
_CHPAD = _NC * _NS * _CPW + _G
_ZROWS = 79
_NPAD = 10112
_ZPT = _NPAD // _NS
_BM = 1000


def _sc_segsum(width, name):
    mesh = plsc.VectorSubcoreMesh(core_axis_name="c", subcore_axis_name="s")

    @functools.partial(
        pl.kernel,
        mesh=mesh,
        name=name,
        out_type=jax.ShapeDtypeStruct((_NC, _NPAD, width), jnp.float32),
        scratch_types=[
            pltpu.VMEM((_G, _CH), jnp.int32),
            pltpu.VMEM((_G, _CH), jnp.int32),
            pltpu.VMEM((_CH, _WA), jnp.float32),
            pltpu.VMEM((_CH, _WA), jnp.float32),
            pltpu.VMEM((_ZROWS, _CH), jnp.float32),
            pltpu.VMEM_SHARED((_NPAD, _WA), jnp.float32),
            pltpu.SemaphoreType.DMA,
            pltpu.SemaphoreType.DMA,
        ],
    )
    def k(feat_hbm, src_hbm, dst_hbm, out_hbm,
          src_v, dst_v, rows0_v, rows1_v, zero_v, acc_sh, sem0, sem1):
        c = lax.axis_index("c")
        s = lax.axis_index("s")
        wid = c * _NS + s
        base = wid * _CPW
        nch = jnp.clip(_RCH - base, 0, _CPW)

        def zfill(i, carry):
            zero_v[i >> 3, pl.ds((i & 7) * 16, 16)] = jnp.zeros(
                (16,), jnp.float32)
            return carry

        lax.fori_loop(0, _ZROWS * 8, zfill, 0)
        for q in range(_ZPT // _ZROWS):
            pltpu.sync_copy(zero_v,
                            acc_sh.at[pl.ds(s * _ZPT + q * _ZROWS, _ZROWS)])
        plsc.subcore_barrier()

        rows = (rows0_v, rows1_v)
        sems = (sem0, sem1)

        def group_body(gi, carry):
            gbase = base + gi * _G
            pltpu.sync_copy(src_hbm.at[pl.ds(gbase, _G)], src_v)
            pltpu.sync_copy(dst_hbm.at[pl.ds(gbase, _G)], dst_v)
            nc = jnp.clip(nch - gi * _G, 0, _G)
            pltpu.async_copy(feat_hbm.at[src_v.at[0]], rows0_v, sem0)
            pltpu.async_copy(feat_hbm.at[src_v.at[1]], rows1_v, sem1)

            def pair_body(g, carry2):
                for b in range(2):
                    kc = 2 * g + b
                    pltpu.make_async_copy(
                        feat_hbm.at[src_v.at[kc]], rows[b], sems[b]).wait()
                    pltpu.sync_copy(rows[b], acc_sh.at[dst_v.at[kc]],
                                    add=True)
                    pltpu.async_copy(
                        feat_hbm.at[src_v.at[kc + 2]], rows[b], sems[b])
                return carry2

            lax.fori_loop(0, (nc - 2) // 2, pair_body, carry)
            for b in range(2):
                kc = nc - 2 + b
                pltpu.make_async_copy(
                    feat_hbm.at[src_v.at[kc]], rows[b], sems[b]).wait()
                pltpu.sync_copy(rows[b], acc_sh.at[dst_v.at[kc]], add=True)
            return carry

        lax.fori_loop(0, (nch + _G - 1) // _G, group_body, 0)
        plsc.subcore_barrier()
        pltpu.sync_copy(acc_sh.at[pl.ds(s * _ZPT, _ZPT)],
                        out_hbm.at[c, pl.ds(s * _ZPT, _ZPT)])

    return k


def _tc1_body(x_ref, wl_ref, wr_ref, b_ref, aug_ref, r1_ref):
    xb = x_ref[...]
    p = jnp.dot(xb, wl_ref[...], preferred_element_type=jnp.float32)
    ones = jnp.ones((_BM, 1), jnp.float32)
    zpad = jnp.zeros((_BM, _WA - _H - 1), jnp.float32)
    aug_ref[...] = jnp.concatenate([p, ones, zpad], axis=1)
    r1_ref[...] = (jnp.dot(xb, wr_ref[...], preferred_element_type=jnp.float32)
                   + b_ref[...])


def _tc2_body(aa_ref, ab_ref, r1_ref, wl_ref, wr_ref, b_ref,
              p2_ref, r2_ref, inv_ref):
    agg = aa_ref[0] + ab_ref[0]
    inv = 1.0 / jnp.maximum(agg[:, _H:_H + 1], 1.0)
    h = jnp.maximum(agg[:, :_H] * inv + r1_ref[...], 0.0)
    p2 = jnp.dot(h, wl_ref[...], preferred_element_type=jnp.float32)
    p2_ref[...] = jnp.concatenate(
        [p2, jnp.zeros((_BM, _WA - _H2), jnp.float32)], axis=1)
    r2_ref[...] = (jnp.dot(h, wr_ref[...], preferred_element_type=jnp.float32)
                   + b_ref[...])
    inv_ref[...] = inv


def _tc3_body(aa_ref, ab_ref, inv_ref, r2_ref, wh_ref, bh_ref, out_ref):
    agg = aa_ref[0, :, :_H2] + ab_ref[0, :, :_H2]
    h2 = jnp.maximum(agg * inv_ref[...] + r2_ref[...], 0.0)
    out_ref[...] = (jnp.dot(h2, wh_ref[...], preferred_element_type=jnp.float32)
                    + bh_ref[...])


def _rows(i):
    return (i, 0)


def _rep(i):
    return (0, 0)


def kernel(x, edge_index, W1l, b1, W1r, W2l, b2, W2r, Wh, bh):
    grid = (_N // _BM,)
    padch = jnp.zeros((_CHPAD - _RCH, _CH), jnp.int32)
    src = jnp.concatenate([edge_index[0].reshape(_RCH, _CH), padch])
    dst = jnp.concatenate([edge_index[1].reshape(_RCH, _CH), padch])

    aug, r1 = pl.pallas_call(
        _tc1_body,
        grid=grid,
        in_specs=[
            pl.BlockSpec((_BM, _D), _rows),
            pl.BlockSpec((_D, _H), _rep),
            pl.BlockSpec((_D, _H), _rep),
            pl.BlockSpec((1, _H), _rep),
        ],
        out_specs=[
            pl.BlockSpec((_BM, _WA), _rows),
            pl.BlockSpec((_BM, _H), _rows),
        ],
        out_shape=[
            jax.ShapeDtypeStruct((_N, _WA), jnp.float32),
            jax.ShapeDtypeStruct((_N, _H), jnp.float32),
        ],
    )(x, W1l, W1r, b1.reshape(1, _H))

    agg1 = _sc_segsum(_WA, "sc_agg1")(aug, src, dst)

    p2, r2, inv = pl.pallas_call(
        _tc2_body,
        grid=grid,
        in_specs=[
            pl.BlockSpec((1, _BM, _WA), lambda i: (0, i, 0)),
            pl.BlockSpec((1, _BM, _WA), lambda i: (1, i, 0)),
            pl.BlockSpec((_BM, _H), _rows),
            pl.BlockSpec((_H, _H2), _rep),
            pl.BlockSpec((_H, _H2), _rep),
            pl.BlockSpec((1, _H2), _rep),
        ],
        out_specs=[
            pl.BlockSpec((_BM, _WA), _rows),
            pl.BlockSpec((_BM, _H2), _rows),
            pl.BlockSpec((_BM, 1), _rows),
        ],
        out_shape=[
            jax.ShapeDtypeStruct((_N, _WA), jnp.float32),
            jax.ShapeDtypeStruct((_N, _H2), jnp.float32),
            jax.ShapeDtypeStruct((_N, 1), jnp.float32),
        ],
    )(agg1, agg1, r1, W2l, W2r, b2.reshape(1, _H2))

    agg2 = _sc_segsum(_WA, "sc_agg2")(p2, src, dst)

    out = pl.pallas_call(
        _tc3_body,
        grid=grid,
        in_specs=[
            pl.BlockSpec((1, _BM, _WA), lambda i: (0, i, 0)),
            pl.BlockSpec((1, _BM, _WA), lambda i: (1, i, 0)),
            pl.BlockSpec((_BM, 1), _rows),
            pl.BlockSpec((_BM, _H2), _rows),
            pl.BlockSpec((_H2, 1), _rep),
            pl.BlockSpec((1, 1), _rep),
        ],
        out_specs=pl.BlockSpec((_BM, 1), _rows),
        out_shape=jax.ShapeDtypeStruct((_N, 1), jnp.float32),
    )(agg2, agg2, inv, r2, Wh, bh.reshape(1, 1))

    return out[:, 0]
